# Initial kernel scaffold; baseline (speedup 1.0000x reference)
import jax
import jax.numpy as jnp
from jax.experimental import pallas as pl


def kernel(offense_ids, defense_ids, offense_pos, defense_pos, gamestate, off_embed, def_embed, bias, gs_w):
    raise NotImplementedError("write your pallas kernel here")


# speedup vs baseline: 1.8968x; 1.8968x over previous
import functools

import jax
import jax.numpy as jnp
from jax import lax
from jax.experimental import pallas as pl
from jax.experimental.pallas import tpu as pltpu
from jax.experimental.pallas import tpu_sc as plsc

_NC = 2
_NS = 16
_LANES = 16
_HALF = _NS // 2
_NGROUPS = _NC * _HALF
_NW = _NC * _NS


@functools.lru_cache(maxsize=None)
def _build(B, L, GS, NP):
    bpg = B // _NGROUPS
    nch = bpg // _LANES
    kh = GS // 2
    bpt = B // _NW
    ncho = bpt // _LANES

    mesh = plsc.VectorSubcoreMesh(
        core_axis_name="c", subcore_axis_name="s",
        num_cores=_NC, num_subcores=_NS)

    @functools.partial(
        pl.kernel,
        out_type=jax.ShapeDtypeStruct((B,), jnp.float32),
        mesh=mesh,
        compiler_params=pltpu.CompilerParams(needs_layout_passes=False),
        scratch_types=[
            pltpu.VMEM((NP,), jnp.float32),
            pltpu.VMEM((L * bpg,), jnp.int32),
            pltpu.VMEM((kh * bpg,), jnp.float32),
            pltpu.VMEM((GS + 8,), jnp.float32),
            pltpu.VMEM((bpg,), jnp.float32),
            pltpu.VMEM((bpt,), jnp.float32),
            pltpu.VMEM((bpt,), jnp.float32),
            pltpu.VMEM_SHARED((_NS * bpg,), jnp.float32),
            pltpu.SemaphoreType.DMA,
        ],
    )
    def run(tabs_h, ids_h, gst_h, par_h, out_h,
            tab_v, ids_v, gs_v, par_v, psum_v, po_v, pd_v, shared, sem):
        c = lax.axis_index("c")
        s = lax.axis_index("s")
        is_off = s < _HALF
        g = lax.rem(s, _HALF)
        base = (c * _HALF + g) * bpg

        toffs = jnp.where(is_off, 0, NP)
        ioffs = jnp.where(is_off, 0, L * B)
        koffs = jnp.where(is_off, 0, kh)

        pltpu.async_copy(tabs_h.at[pl.ds(toffs, NP)], tab_v, sem)
        for l in range(L):
            pltpu.sync_copy(ids_h.at[pl.ds(ioffs + l * B + base, bpg)],
                            ids_v.at[pl.ds(l * bpg, bpg)])
        for k in range(kh):
            pltpu.sync_copy(gst_h.at[pl.ds((koffs + k) * B + base, bpg)],
                            gs_v.at[pl.ds(k * bpg, bpg)])
        pltpu.sync_copy(par_h, par_v)

        wk = [plsc.load_gather(par_v, [jnp.full((_LANES,), k, jnp.int32) + koffs])
              for k in range(kh)]
        b_splat = plsc.load_gather(par_v, [jnp.full((_LANES,), GS, jnp.int32)])
        acc0 = b_splat * is_off.astype(jnp.float32)

        pltpu.make_async_copy(tabs_h.at[pl.ds(toffs, NP)], tab_v, sem).wait()

        def chunk(i, carry):
            col = pl.multiple_of(i * _LANES, _LANES)
            acc = acc0
            for l in range(L):
                acc = acc + plsc.load_gather(
                    tab_v, [ids_v[pl.ds(l * bpg + col, _LANES)]])
            for k in range(kh):
                acc = acc + gs_v[pl.ds(k * bpg + col, _LANES)] * wk[k]
            psum_v[pl.ds(col, _LANES)] = acc
            return carry

        lax.fori_loop(0, nch, chunk, 0)

        pltpu.sync_copy(psum_v, shared.at[pl.ds(s * bpg, bpg)])
        plsc.subcore_barrier()

        go = s // 2
        ho = lax.rem(s, 2) * bpt
        pltpu.sync_copy(shared.at[pl.ds(go * bpg + ho, bpt)], po_v)
        pltpu.sync_copy(shared.at[pl.ds((_HALF + go) * bpg + ho, bpt)], pd_v)

        def fold(i, carry):
            col = pl.multiple_of(i * _LANES, _LANES)
            po_v[pl.ds(col, _LANES)] = (
                po_v[pl.ds(col, _LANES)] + pd_v[pl.ds(col, _LANES)])
            return carry

        lax.fori_loop(0, ncho, fold, 0)
        obase = c * _HALF * bpg + s * bpt
        pltpu.sync_copy(po_v, out_h.at[pl.ds(obase, bpt)])

    return run


def kernel(offense_ids, defense_ids, offense_pos, defense_pos, gamestate,
           off_embed, def_embed, bias, gs_w):
    del offense_pos, defense_pos
    B, L = offense_ids.shape
    NP = off_embed.shape[0]
    GS = gamestate.shape[1]
    run = _build(B, L, GS, NP)
    tabs = jnp.concatenate(
        [off_embed.reshape(-1), def_embed.reshape(-1)]).astype(jnp.float32)
    ids = jnp.concatenate(
        [offense_ids.T.reshape(-1), defense_ids.T.reshape(-1)]
    ).astype(jnp.int32)
    params = jnp.concatenate(
        [gs_w.reshape(-1), bias.reshape(-1),
         jnp.zeros((7,), jnp.float32)]).astype(jnp.float32)
    return run(tabs, ids, gamestate.T.astype(jnp.float32).reshape(-1), params)
